# Initial kernel scaffold; baseline (speedup 1.0000x reference)
import jax
import jax.numpy as jnp
from jax.experimental import pallas as pl


def kernel(x, embeddings):
    raise NotImplementedError("write your pallas kernel here")


# speedup vs baseline: 1.4017x; 1.4017x over previous
import functools

import jax
import jax.numpy as jnp
from jax import lax
from jax.experimental import pallas as pl
from jax.experimental.pallas import tpu as pltpu
from jax.experimental.pallas import tpu_sc as plsc

E_DIM = 256
N_EMB = 8192
N_ROWS = 16384

BM = 512
WN = 2816
NW = 3
NPAD = WN * NW


def _argmin_body(x_ref, e_ref, x2_ref, e2_ref, idx_out, val_sc, idx_sc):
    w = pl.program_id(1)
    xb = x_ref[...].astype(jnp.bfloat16)
    eb = e_ref[...].astype(jnp.bfloat16)
    sim = jnp.dot(xb, eb, preferred_element_type=jnp.float32)
    dist = (x2_ref[...] + e2_ref[...]) - 2.0 * sim
    vw = jnp.min(dist, axis=1, keepdims=True)
    col = lax.broadcasted_iota(jnp.int32, (BM, WN), 1) + w * WN
    iw = jnp.min(jnp.where(dist == vw, col, jnp.int32(2**30)), axis=1,
                 keepdims=True)
    vq = vw.astype(jnp.bfloat16).astype(jnp.float32)

    @pl.when(w == 0)
    def _():
        val_sc[...] = vq
        idx_sc[...] = iw

    @pl.when(w > 0)
    def _():
        win = vw < val_sc[...]
        val_sc[...] = jnp.where(win, vq, val_sc[...])
        idx_sc[...] = jnp.where(win, iw, idx_sc[...])

    @pl.when(w == NW - 1)
    def _():
        idx_out[...] = idx_sc[...]


_argmin_call = pl.pallas_call(
    _argmin_body,
    grid=(N_ROWS // BM, NW),
    in_specs=[
        pl.BlockSpec((BM, E_DIM), lambda m, w: (m, 0)),
        pl.BlockSpec((E_DIM, WN), lambda m, w: (0, w)),
        pl.BlockSpec((BM, 1), lambda m, w: (m, 0)),
        pl.BlockSpec((1, WN), lambda m, w: (0, w)),
    ],
    out_specs=pl.BlockSpec((BM, 1), lambda m, w: (m, 0)),
    out_shape=jax.ShapeDtypeStruct((N_ROWS, 1), jnp.int32),
    scratch_shapes=[
        pltpu.VMEM((BM, 1), jnp.float32),
        pltpu.VMEM((BM, 1), jnp.int32),
    ],
    compiler_params=pltpu.CompilerParams(
        dimension_semantics=("parallel", "arbitrary"),
    ),
)


def _make_gather():
    info = plsc.get_sparse_core_info()
    nc, ns = info.num_cores, info.num_subcores
    nw = nc * ns
    b_per_w = N_ROWS // nw
    n_chunks = 2
    chunk = b_per_w // n_chunks
    mesh = plsc.VectorSubcoreMesh(core_axis_name="c", subcore_axis_name="s")

    @functools.partial(
        pl.kernel,
        mesh=mesh,
        out_type=jax.ShapeDtypeStruct((N_ROWS, E_DIM), jnp.float32),
        scratch_types=[
            pltpu.VMEM((b_per_w,), jnp.int32),
            pltpu.VMEM((chunk, E_DIM), jnp.float32),
            pltpu.SemaphoreType.DMA,
        ],
    )
    def gather(table_hbm, idx_hbm, out_hbm, idx_v, rows_v, sem):
        wid = lax.axis_index("s") * nc + lax.axis_index("c")
        base = wid * b_per_w
        pltpu.sync_copy(idx_hbm.at[pl.ds(base, b_per_w)], idx_v)
        for c in range(n_chunks):
            idx_c = idx_v.at[pl.ds(c * chunk, chunk)]
            pltpu.async_copy(table_hbm.at[idx_c], rows_v, sem).wait()
            pltpu.sync_copy(rows_v, out_hbm.at[pl.ds(base + c * chunk, chunk)])

    return gather


@functools.cache
def _gather_call():
    return _make_gather()


def kernel(x, embeddings):
    flat = jnp.reshape(x, (N_ROWS, E_DIM))
    x2 = jnp.sum(flat**2, axis=1, keepdims=True)
    e2 = jnp.sum(embeddings**2, axis=0)
    e_pad = jnp.pad(embeddings, ((0, 0), (0, NPAD - N_EMB)))
    e2_pad = jnp.pad(e2, (0, NPAD - N_EMB), constant_values=1e30)[None, :]
    idx = _argmin_call(flat, e_pad, x2, e2_pad)
    table = embeddings.T.astype(jnp.bfloat16).astype(jnp.float32)
    q = _gather_call()(table, jnp.reshape(idx, (N_ROWS,)))
    return jnp.reshape(q, x.shape)


# speedup vs baseline: 1.4044x; 1.0019x over previous
import functools

import jax
import jax.numpy as jnp
from jax import lax
from jax.experimental import pallas as pl
from jax.experimental.pallas import tpu as pltpu
from jax.experimental.pallas import tpu_sc as plsc

E_DIM = 256
N_EMB = 8192
N_ROWS = 16384

BM = 512
WN = 2816
NW = 3
NPAD = WN * NW


def _argmin_body(x_ref, e_ref, x2_ref, e2_ref, idx_out, val_sc, idx_sc):
    w = pl.program_id(0)
    m = pl.program_id(1)
    rows = pl.ds(m * BM, BM)
    xb = x_ref[...].astype(jnp.bfloat16)
    eb = e_ref[...].astype(jnp.bfloat16)
    sim = jnp.dot(xb, eb, preferred_element_type=jnp.float32)
    dist = (x2_ref[...] + e2_ref[...]) - 2.0 * sim
    vw = jnp.min(dist, axis=1, keepdims=True)
    col = lax.broadcasted_iota(jnp.int32, (BM, WN), 1) + w * WN
    iw = jnp.min(jnp.where(dist == vw, col, jnp.int32(2**30)), axis=1,
                 keepdims=True)
    vq = vw.astype(jnp.bfloat16).astype(jnp.float32)

    @pl.when(w == 0)
    def _():
        val_sc[rows, :] = vq
        idx_sc[rows, :] = iw

    @pl.when(w > 0)
    def _():
        win = vw < val_sc[rows, :]
        val_sc[rows, :] = jnp.where(win, vq, val_sc[rows, :])
        idx_sc[rows, :] = jnp.where(win, iw, idx_sc[rows, :])

    @pl.when(w == NW - 1)
    def _():
        idx_out[...] = idx_sc[rows, :]


_argmin_call = pl.pallas_call(
    _argmin_body,
    grid=(NW, N_ROWS // BM),
    in_specs=[
        pl.BlockSpec((BM, E_DIM), lambda w, m: (m, 0)),
        pl.BlockSpec((E_DIM, WN), lambda w, m: (0, w)),
        pl.BlockSpec((BM, 1), lambda w, m: (m, 0)),
        pl.BlockSpec((1, WN), lambda w, m: (0, w)),
    ],
    out_specs=pl.BlockSpec((BM, 1), lambda w, m: (m, 0)),
    out_shape=jax.ShapeDtypeStruct((N_ROWS, 1), jnp.int32),
    scratch_shapes=[
        pltpu.VMEM((N_ROWS, 1), jnp.float32),
        pltpu.VMEM((N_ROWS, 1), jnp.int32),
    ],
    compiler_params=pltpu.CompilerParams(
        dimension_semantics=("arbitrary", "arbitrary"),
    ),
)


def _make_gather():
    info = plsc.get_sparse_core_info()
    nc, ns = info.num_cores, info.num_subcores
    nw = nc * ns
    b_per_w = N_ROWS // nw
    n_chunks = 2
    chunk = b_per_w // n_chunks
    mesh = plsc.VectorSubcoreMesh(core_axis_name="c", subcore_axis_name="s")

    @functools.partial(
        pl.kernel,
        mesh=mesh,
        out_type=jax.ShapeDtypeStruct((N_ROWS, E_DIM), jnp.float32),
        scratch_types=[
            pltpu.VMEM((b_per_w,), jnp.int32),
            pltpu.VMEM((chunk, E_DIM), jnp.float32),
            pltpu.SemaphoreType.DMA,
        ],
    )
    def gather(table_hbm, idx_hbm, out_hbm, idx_v, rows_v, sem):
        wid = lax.axis_index("s") * nc + lax.axis_index("c")
        base = wid * b_per_w
        pltpu.sync_copy(idx_hbm.at[pl.ds(base, b_per_w)], idx_v)
        for c in range(n_chunks):
            idx_c = idx_v.at[pl.ds(c * chunk, chunk)]
            pltpu.async_copy(table_hbm.at[idx_c], rows_v, sem).wait()
            pltpu.sync_copy(rows_v, out_hbm.at[pl.ds(base + c * chunk, chunk)])

    return gather


@functools.cache
def _gather_call():
    return _make_gather()


def kernel(x, embeddings):
    flat = jnp.reshape(x, (N_ROWS, E_DIM))
    x2 = jnp.sum(flat**2, axis=1, keepdims=True)
    e2 = jnp.sum(embeddings**2, axis=0)
    e_pad = jnp.pad(embeddings, ((0, 0), (0, NPAD - N_EMB)))
    e2_pad = jnp.pad(e2, (0, NPAD - N_EMB), constant_values=1e30)[None, :]
    idx = _argmin_call(flat, e_pad, x2, e2_pad)
    table = embeddings.T.astype(jnp.bfloat16).astype(jnp.float32)
    q = _gather_call()(table, jnp.reshape(idx, (N_ROWS,)))
    return jnp.reshape(q, x.shape)
